# baseline (speedup 1.0000x reference)
import functools

import jax
import jax.numpy as jnp
from jax.experimental import pallas as pl
from jax.experimental.pallas import tpu as pltpu

_B, _N, _IN_C, _OUT_C, _K = 4, 2048, 128, 256, 16
_RB = 256
_RB2 = 1024
_BN = _B * _N


def _gelu(x):
    return 0.5 * x * (1.0 + jax.lax.erf(x * 0.7071067811865476))


def _bdot(a, b):
    return jnp.dot(a.astype(jnp.bfloat16), b.astype(jnp.bfloat16),
                   preferred_element_type=jnp.float32)


def _layernorm(x, g, b):
    m = jnp.mean(x, axis=-1, keepdims=True)
    v = jnp.mean((x - m) * (x - m), axis=-1, keepdims=True)
    return (x - m) / jnp.sqrt(v + 1e-5) * g + b


def _knn_pool_body(pts_ref, ptst_ref, feats_ref, x_ref, d_scr):
    ptsr = pts_ref[0]
    ptst = ptst_ref[0]
    sq_r = jnp.sum(ptsr * ptsr, axis=1, keepdims=True)
    sq_c = jnp.sum(ptst * ptst, axis=0, keepdims=True)
    pr = ptsr.astype(jnp.bfloat16).astype(jnp.float32)
    pt = ptst.astype(jnp.bfloat16).astype(jnp.float32)
    prod = (pr[:, 0:1] * pt[0:1, :]
            + pr[:, 1:2] * pt[1:2, :]
            + pr[:, 2:3] * pt[2:3, :])
    d2 = sq_r + sq_c - 2.0 * prod
    d_scr[...] = jnp.sqrt(jnp.maximum(d2, 0.0))
    iota = jax.lax.broadcasted_iota(jnp.int32, (_RB, _N), 1)

    def body(_, carry):
        dd = d_scr[...]
        sel = jnp.argmin(dd, axis=1).reshape(_RB, 1)
        d_scr[...] = jnp.where(iota == sel, jnp.float32(jnp.inf), dd)
        return carry

    jax.lax.fori_loop(0, _K, body, 0)
    mask = jnp.where(jnp.isinf(d_scr[...]), 1.0, 0.0).astype(jnp.bfloat16)
    x_ref[0] = jnp.dot(mask, feats_ref[0].astype(jnp.bfloat16),
                       preferred_element_type=jnp.float32) * (1.0 / _K)


def _stage_a(pts, pts_t, feats):
    return pl.pallas_call(
        _knn_pool_body,
        grid=(_B, _N // _RB),
        in_specs=[
            pl.BlockSpec((1, _RB, 3), lambda b, r: (b, r, 0)),
            pl.BlockSpec((1, 3, _N), lambda b, r: (b, 0, 0)),
            pl.BlockSpec((1, _N, _IN_C), lambda b, r: (b, 0, 0)),
        ],
        out_specs=pl.BlockSpec((1, _RB, _IN_C), lambda b, r: (b, r, 0)),
        out_shape=jax.ShapeDtypeStruct((_B, _N, _IN_C), jnp.float32),
        scratch_shapes=[
            pltpu.VMEM((_RB, _N), jnp.float32),
        ],
    )(pts, pts_t, feats)


def _dense1_body(x_ref, fc1_w_ref, fc1_b_ref, ln1_g_ref, ln1_b_ref,
                 fc2_w_ref, fc2_b_ref, ln2_g_ref, ln2_b_ref,
                 dw_w_ref, dw_b_ref, pw_w_ref, pw_b_ref,
                 y_ref, stats_ref):
    x = x_ref[...]
    h = _bdot(x, fc1_w_ref[...]) + fc1_b_ref[...]
    h = _layernorm(_gelu(h), ln1_g_ref[...], ln1_b_ref[...])
    h = _bdot(h, fc2_w_ref[...]) + fc2_b_ref[...]
    h = _layernorm(_gelu(h), ln2_g_ref[...], ln2_b_ref[...])
    y = h * dw_w_ref[...] + dw_b_ref[...]
    y = _bdot(y, pw_w_ref[...]) + pw_b_ref[...]
    y_ref[...] = y

    @pl.when(pl.program_id(0) == 0)
    def _init():
        stats_ref[...] = jnp.zeros((8, _OUT_C), jnp.float32)

    stats_ref[0:1, :] += jnp.sum(y, axis=0, keepdims=True)
    stats_ref[1:2, :] += jnp.sum(y * y, axis=0, keepdims=True)


def _stage_b1(x2, fc1_w, fc1_b, ln1_g, ln1_b, fc2_w, fc2_b, ln2_g, ln2_b,
              dw_w, dw_b, pw_w, pw_b):
    row_spec = pl.BlockSpec((_RB2, _IN_C), lambda i: (i, 0))
    wfull = lambda shape: pl.BlockSpec(shape, lambda i: (0, 0))
    return pl.pallas_call(
        _dense1_body,
        grid=(_BN // _RB2,),
        in_specs=[
            row_spec,
            wfull((_IN_C, _OUT_C)), wfull((1, _OUT_C)),
            wfull((1, _OUT_C)), wfull((1, _OUT_C)),
            wfull((_OUT_C, _OUT_C)), wfull((1, _OUT_C)),
            wfull((1, _OUT_C)), wfull((1, _OUT_C)),
            wfull((1, _OUT_C)), wfull((1, _OUT_C)),
            wfull((_OUT_C, _OUT_C)), wfull((1, _OUT_C)),
        ],
        out_specs=[
            pl.BlockSpec((_RB2, _OUT_C), lambda i: (i, 0)),
            pl.BlockSpec((8, _OUT_C), lambda i: (0, 0)),
        ],
        out_shape=[
            jax.ShapeDtypeStruct((_BN, _OUT_C), jnp.float32),
            jax.ShapeDtypeStruct((8, _OUT_C), jnp.float32),
        ],
    )(x2, fc1_w, fc1_b, ln1_g, ln1_b, fc2_w, fc2_b, ln2_g, ln2_b,
      dw_w, dw_b, pw_w, pw_b)


def _dense2_body(y_ref, stats_ref, bn_g_ref, bn_b_ref,
                 feats_ref, proj_w_ref, proj_b_ref, out_ref):
    m = stats_ref[0:1, :] * (1.0 / _BN)
    v = stats_ref[1:2, :] * (1.0 / _BN) - m * m
    y = (y_ref[...] - m) / jnp.sqrt(v + 1e-5) * bn_g_ref[...] + bn_b_ref[...]
    y = _gelu(y)
    proj = _bdot(feats_ref[...], proj_w_ref[...]) + proj_b_ref[...]
    out_ref[...] = y + proj


def _stage_b2(y_pre, stats, bn_g, bn_b, feats2, proj_w, proj_b):
    wfull = lambda shape: pl.BlockSpec(shape, lambda i: (0, 0))
    return pl.pallas_call(
        _dense2_body,
        grid=(_BN // _RB2,),
        in_specs=[
            pl.BlockSpec((_RB2, _OUT_C), lambda i: (i, 0)),
            wfull((8, _OUT_C)),
            wfull((1, _OUT_C)), wfull((1, _OUT_C)),
            pl.BlockSpec((_RB2, _IN_C), lambda i: (i, 0)),
            wfull((_IN_C, _OUT_C)), wfull((1, _OUT_C)),
        ],
        out_specs=pl.BlockSpec((_RB2, _OUT_C), lambda i: (i, 0)),
        out_shape=jax.ShapeDtypeStruct((_BN, _OUT_C), jnp.float32),
    )(y_pre, stats, bn_g, bn_b, feats2, proj_w, proj_b)


def kernel(pts, feats, fc1_w, fc1_b, ln1_g, ln1_b, fc2_w, fc2_b, ln2_g, ln2_b,
           dw_w, dw_b, pw_w, pw_b, bn_g, bn_b, proj_w, proj_b):
    pts_t = jnp.transpose(pts, (0, 2, 1))
    x = _stage_a(pts, pts_t, feats)
    x2 = x.reshape(_BN, _IN_C)
    f2 = feats.reshape(_BN, _IN_C)
    r = lambda w: w.reshape(1, _OUT_C)
    y_pre, stats = _stage_b1(x2, fc1_w, r(fc1_b), r(ln1_g), r(ln1_b),
                             fc2_w, r(fc2_b), r(ln2_g), r(ln2_b),
                             r(dw_w), r(dw_b), pw_w, r(pw_b))
    out2 = _stage_b2(y_pre, stats, r(bn_g), r(bn_b), f2, proj_w, r(proj_b))
    return out2.reshape(_B, _N, _OUT_C)


# speedup vs baseline: 1.2755x; 1.2755x over previous
import functools

import jax
import jax.numpy as jnp
from jax.experimental import pallas as pl
from jax.experimental.pallas import tpu as pltpu

_B, _N, _IN_C, _OUT_C, _K = 4, 2048, 128, 256, 16
_RB = 512
_RB2 = 1024
_BN = _B * _N


def _gelu(x):
    return 0.5 * x * (1.0 + jax.lax.erf(x * 0.7071067811865476))


def _bdot(a, b):
    return jnp.dot(a.astype(jnp.bfloat16), b.astype(jnp.bfloat16),
                   preferred_element_type=jnp.float32)


def _layernorm(x, g, b):
    m = jnp.mean(x, axis=-1, keepdims=True)
    v = jnp.mean((x - m) * (x - m), axis=-1, keepdims=True)
    return (x - m) / jnp.sqrt(v + 1e-5) * g + b


def _knn_pool_body(pts_ref, ptst_ref, feats_ref, x_ref, d_scr):
    ptsr = pts_ref[0]
    ptst = ptst_ref[0]
    sq_r = jnp.sum(ptsr * ptsr, axis=1, keepdims=True)
    sq_c = jnp.sum(ptst * ptst, axis=0, keepdims=True)
    pr = ptsr.astype(jnp.bfloat16).astype(jnp.float32)
    pt = ptst.astype(jnp.bfloat16).astype(jnp.float32)
    prod = (pr[:, 0:1] * pt[0:1, :]
            + pr[:, 1:2] * pt[1:2, :]
            + pr[:, 2:3] * pt[2:3, :])
    d2 = sq_r + sq_c - 2.0 * prod
    d_scr[...] = jnp.sqrt(jnp.maximum(d2, 0.0))
    iota = jax.lax.broadcasted_iota(jnp.int32, (_RB, _N), 1)

    def body(_, carry):
        dd = d_scr[...]
        m = jnp.min(dd, axis=1, keepdims=True)
        sel = jnp.min(jnp.where(dd == m, iota, _N), axis=1, keepdims=True)
        d_scr[...] = jnp.where(iota == sel, jnp.float32(jnp.inf), dd)
        return carry

    jax.lax.fori_loop(0, _K, body, 0)
    mask = jnp.where(jnp.isinf(d_scr[...]), 1.0, 0.0).astype(jnp.bfloat16)
    x_ref[0] = jnp.dot(mask, feats_ref[0].astype(jnp.bfloat16),
                       preferred_element_type=jnp.float32) * (1.0 / _K)


def _stage_a(pts, pts_t, feats):
    return pl.pallas_call(
        _knn_pool_body,
        grid=(_B, _N // _RB),
        in_specs=[
            pl.BlockSpec((1, _RB, 3), lambda b, r: (b, r, 0)),
            pl.BlockSpec((1, 3, _N), lambda b, r: (b, 0, 0)),
            pl.BlockSpec((1, _N, _IN_C), lambda b, r: (b, 0, 0)),
        ],
        out_specs=pl.BlockSpec((1, _RB, _IN_C), lambda b, r: (b, r, 0)),
        out_shape=jax.ShapeDtypeStruct((_B, _N, _IN_C), jnp.float32),
        scratch_shapes=[
            pltpu.VMEM((_RB, _N), jnp.float32),
        ],
    )(pts, pts_t, feats)


def _dense1_body(x_ref, fc1_w_ref, fc1_b_ref, ln1_g_ref, ln1_b_ref,
                 fc2_w_ref, fc2_b_ref, ln2_g_ref, ln2_b_ref,
                 dw_w_ref, dw_b_ref, pw_w_ref, pw_b_ref,
                 y_ref, stats_ref):
    x = x_ref[...]
    h = _bdot(x, fc1_w_ref[...]) + fc1_b_ref[...]
    h = _layernorm(_gelu(h), ln1_g_ref[...], ln1_b_ref[...])
    h = _bdot(h, fc2_w_ref[...]) + fc2_b_ref[...]
    h = _layernorm(_gelu(h), ln2_g_ref[...], ln2_b_ref[...])
    y = h * dw_w_ref[...] + dw_b_ref[...]
    y = _bdot(y, pw_w_ref[...]) + pw_b_ref[...]
    y_ref[...] = y

    @pl.when(pl.program_id(0) == 0)
    def _init():
        stats_ref[...] = jnp.zeros((8, _OUT_C), jnp.float32)

    stats_ref[0:1, :] += jnp.sum(y, axis=0, keepdims=True)
    stats_ref[1:2, :] += jnp.sum(y * y, axis=0, keepdims=True)


def _stage_b1(x2, fc1_w, fc1_b, ln1_g, ln1_b, fc2_w, fc2_b, ln2_g, ln2_b,
              dw_w, dw_b, pw_w, pw_b):
    row_spec = pl.BlockSpec((_RB2, _IN_C), lambda i: (i, 0))
    wfull = lambda shape: pl.BlockSpec(shape, lambda i: (0, 0))
    return pl.pallas_call(
        _dense1_body,
        grid=(_BN // _RB2,),
        in_specs=[
            row_spec,
            wfull((_IN_C, _OUT_C)), wfull((1, _OUT_C)),
            wfull((1, _OUT_C)), wfull((1, _OUT_C)),
            wfull((_OUT_C, _OUT_C)), wfull((1, _OUT_C)),
            wfull((1, _OUT_C)), wfull((1, _OUT_C)),
            wfull((1, _OUT_C)), wfull((1, _OUT_C)),
            wfull((_OUT_C, _OUT_C)), wfull((1, _OUT_C)),
        ],
        out_specs=[
            pl.BlockSpec((_RB2, _OUT_C), lambda i: (i, 0)),
            pl.BlockSpec((8, _OUT_C), lambda i: (0, 0)),
        ],
        out_shape=[
            jax.ShapeDtypeStruct((_BN, _OUT_C), jnp.float32),
            jax.ShapeDtypeStruct((8, _OUT_C), jnp.float32),
        ],
    )(x2, fc1_w, fc1_b, ln1_g, ln1_b, fc2_w, fc2_b, ln2_g, ln2_b,
      dw_w, dw_b, pw_w, pw_b)


def _dense2_body(y_ref, stats_ref, bn_g_ref, bn_b_ref,
                 feats_ref, proj_w_ref, proj_b_ref, out_ref):
    m = stats_ref[0:1, :] * (1.0 / _BN)
    v = stats_ref[1:2, :] * (1.0 / _BN) - m * m
    y = (y_ref[...] - m) / jnp.sqrt(v + 1e-5) * bn_g_ref[...] + bn_b_ref[...]
    y = _gelu(y)
    proj = _bdot(feats_ref[...], proj_w_ref[...]) + proj_b_ref[...]
    out_ref[...] = y + proj


def _stage_b2(y_pre, stats, bn_g, bn_b, feats2, proj_w, proj_b):
    wfull = lambda shape: pl.BlockSpec(shape, lambda i: (0, 0))
    return pl.pallas_call(
        _dense2_body,
        grid=(_BN // _RB2,),
        in_specs=[
            pl.BlockSpec((_RB2, _OUT_C), lambda i: (i, 0)),
            wfull((8, _OUT_C)),
            wfull((1, _OUT_C)), wfull((1, _OUT_C)),
            pl.BlockSpec((_RB2, _IN_C), lambda i: (i, 0)),
            wfull((_IN_C, _OUT_C)), wfull((1, _OUT_C)),
        ],
        out_specs=pl.BlockSpec((_RB2, _OUT_C), lambda i: (i, 0)),
        out_shape=jax.ShapeDtypeStruct((_BN, _OUT_C), jnp.float32),
    )(y_pre, stats, bn_g, bn_b, feats2, proj_w, proj_b)


def kernel(pts, feats, fc1_w, fc1_b, ln1_g, ln1_b, fc2_w, fc2_b, ln2_g, ln2_b,
           dw_w, dw_b, pw_w, pw_b, bn_g, bn_b, proj_w, proj_b):
    pts_t = jnp.transpose(pts, (0, 2, 1))
    x = _stage_a(pts, pts_t, feats)
    x2 = x.reshape(_BN, _IN_C)
    f2 = feats.reshape(_BN, _IN_C)
    r = lambda w: w.reshape(1, _OUT_C)
    y_pre, stats = _stage_b1(x2, fc1_w, r(fc1_b), r(ln1_g), r(ln1_b),
                             fc2_w, r(fc2_b), r(ln2_g), r(ln2_b),
                             r(dw_w), r(dw_b), pw_w, r(pw_b))
    out2 = _stage_b2(y_pre, stats, r(bn_g), r(bn_b), f2, proj_w, r(proj_b))
    return out2.reshape(_B, _N, _OUT_C)


# speedup vs baseline: 1.3019x; 1.0206x over previous
import functools

import jax
import jax.numpy as jnp
from jax.experimental import pallas as pl
from jax.experimental.pallas import tpu as pltpu

_B, _N, _IN_C, _OUT_C, _K = 4, 2048, 128, 256, 16
_RB = 1024
_RB2 = 1024
_BN = _B * _N


def _gelu(x):
    return 0.5 * x * (1.0 + jax.lax.erf(x * 0.7071067811865476))


def _bdot(a, b):
    return jnp.dot(a.astype(jnp.bfloat16), b.astype(jnp.bfloat16),
                   preferred_element_type=jnp.float32)


def _layernorm(x, g, b):
    m = jnp.mean(x, axis=-1, keepdims=True)
    v = jnp.mean((x - m) * (x - m), axis=-1, keepdims=True)
    return (x - m) / jnp.sqrt(v + 1e-5) * g + b


def _knn_pool_body(pts_ref, ptst_ref, feats_ref, x_ref, d_scr):
    ptsr = pts_ref[0]
    ptst = ptst_ref[0]
    sq_r = jnp.sum(ptsr * ptsr, axis=1, keepdims=True)
    sq_c = jnp.sum(ptst * ptst, axis=0, keepdims=True)
    pr = ptsr.astype(jnp.bfloat16).astype(jnp.float32)
    pt = ptst.astype(jnp.bfloat16).astype(jnp.float32)
    prod = (pr[:, 0:1] * pt[0:1, :]
            + pr[:, 1:2] * pt[1:2, :]
            + pr[:, 2:3] * pt[2:3, :])
    d2 = sq_r + sq_c - 2.0 * prod
    d_scr[...] = jnp.sqrt(jnp.maximum(d2, 0.0))
    iota = jax.lax.broadcasted_iota(jnp.int32, (_RB, _N), 1)

    def body(_, carry):
        dd = d_scr[...]
        m = jnp.min(dd, axis=1, keepdims=True)
        sel = jnp.min(jnp.where(dd == m, iota, _N), axis=1, keepdims=True)
        d_scr[...] = jnp.where(iota == sel, jnp.float32(jnp.inf), dd)
        return carry

    jax.lax.fori_loop(0, _K, body, 0)
    mask = jnp.where(jnp.isinf(d_scr[...]), 1.0, 0.0).astype(jnp.bfloat16)
    x_ref[0] = jnp.dot(mask, feats_ref[0].astype(jnp.bfloat16),
                       preferred_element_type=jnp.float32) * (1.0 / _K)


def _stage_a(pts, pts_t, feats):
    return pl.pallas_call(
        _knn_pool_body,
        grid=(_B, _N // _RB),
        in_specs=[
            pl.BlockSpec((1, _RB, 3), lambda b, r: (b, r, 0)),
            pl.BlockSpec((1, 3, _N), lambda b, r: (b, 0, 0)),
            pl.BlockSpec((1, _N, _IN_C), lambda b, r: (b, 0, 0)),
        ],
        out_specs=pl.BlockSpec((1, _RB, _IN_C), lambda b, r: (b, r, 0)),
        out_shape=jax.ShapeDtypeStruct((_B, _N, _IN_C), jnp.float32),
        scratch_shapes=[
            pltpu.VMEM((_RB, _N), jnp.float32),
        ],
    )(pts, pts_t, feats)


def _dense1_body(x_ref, fc1_w_ref, fc1_b_ref, ln1_g_ref, ln1_b_ref,
                 fc2_w_ref, fc2_b_ref, ln2_g_ref, ln2_b_ref,
                 dw_w_ref, dw_b_ref, pw_w_ref, pw_b_ref,
                 y_ref, stats_ref):
    x = x_ref[...]
    h = _bdot(x, fc1_w_ref[...]) + fc1_b_ref[...]
    h = _layernorm(_gelu(h), ln1_g_ref[...], ln1_b_ref[...])
    h = _bdot(h, fc2_w_ref[...]) + fc2_b_ref[...]
    h = _layernorm(_gelu(h), ln2_g_ref[...], ln2_b_ref[...])
    y = h * dw_w_ref[...] + dw_b_ref[...]
    y = _bdot(y, pw_w_ref[...]) + pw_b_ref[...]
    y_ref[...] = y

    @pl.when(pl.program_id(0) == 0)
    def _init():
        stats_ref[...] = jnp.zeros((8, _OUT_C), jnp.float32)

    stats_ref[0:1, :] += jnp.sum(y, axis=0, keepdims=True)
    stats_ref[1:2, :] += jnp.sum(y * y, axis=0, keepdims=True)


def _stage_b1(x2, fc1_w, fc1_b, ln1_g, ln1_b, fc2_w, fc2_b, ln2_g, ln2_b,
              dw_w, dw_b, pw_w, pw_b):
    row_spec = pl.BlockSpec((_RB2, _IN_C), lambda i: (i, 0))
    wfull = lambda shape: pl.BlockSpec(shape, lambda i: (0, 0))
    return pl.pallas_call(
        _dense1_body,
        grid=(_BN // _RB2,),
        in_specs=[
            row_spec,
            wfull((_IN_C, _OUT_C)), wfull((1, _OUT_C)),
            wfull((1, _OUT_C)), wfull((1, _OUT_C)),
            wfull((_OUT_C, _OUT_C)), wfull((1, _OUT_C)),
            wfull((1, _OUT_C)), wfull((1, _OUT_C)),
            wfull((1, _OUT_C)), wfull((1, _OUT_C)),
            wfull((_OUT_C, _OUT_C)), wfull((1, _OUT_C)),
        ],
        out_specs=[
            pl.BlockSpec((_RB2, _OUT_C), lambda i: (i, 0)),
            pl.BlockSpec((8, _OUT_C), lambda i: (0, 0)),
        ],
        out_shape=[
            jax.ShapeDtypeStruct((_BN, _OUT_C), jnp.float32),
            jax.ShapeDtypeStruct((8, _OUT_C), jnp.float32),
        ],
    )(x2, fc1_w, fc1_b, ln1_g, ln1_b, fc2_w, fc2_b, ln2_g, ln2_b,
      dw_w, dw_b, pw_w, pw_b)


def _dense2_body(y_ref, stats_ref, bn_g_ref, bn_b_ref,
                 feats_ref, proj_w_ref, proj_b_ref, out_ref):
    m = stats_ref[0:1, :] * (1.0 / _BN)
    v = stats_ref[1:2, :] * (1.0 / _BN) - m * m
    y = (y_ref[...] - m) / jnp.sqrt(v + 1e-5) * bn_g_ref[...] + bn_b_ref[...]
    y = _gelu(y)
    proj = _bdot(feats_ref[...], proj_w_ref[...]) + proj_b_ref[...]
    out_ref[...] = y + proj


def _stage_b2(y_pre, stats, bn_g, bn_b, feats2, proj_w, proj_b):
    wfull = lambda shape: pl.BlockSpec(shape, lambda i: (0, 0))
    return pl.pallas_call(
        _dense2_body,
        grid=(_BN // _RB2,),
        in_specs=[
            pl.BlockSpec((_RB2, _OUT_C), lambda i: (i, 0)),
            wfull((8, _OUT_C)),
            wfull((1, _OUT_C)), wfull((1, _OUT_C)),
            pl.BlockSpec((_RB2, _IN_C), lambda i: (i, 0)),
            wfull((_IN_C, _OUT_C)), wfull((1, _OUT_C)),
        ],
        out_specs=pl.BlockSpec((_RB2, _OUT_C), lambda i: (i, 0)),
        out_shape=jax.ShapeDtypeStruct((_BN, _OUT_C), jnp.float32),
    )(y_pre, stats, bn_g, bn_b, feats2, proj_w, proj_b)


def kernel(pts, feats, fc1_w, fc1_b, ln1_g, ln1_b, fc2_w, fc2_b, ln2_g, ln2_b,
           dw_w, dw_b, pw_w, pw_b, bn_g, bn_b, proj_w, proj_b):
    pts_t = jnp.transpose(pts, (0, 2, 1))
    x = _stage_a(pts, pts_t, feats)
    x2 = x.reshape(_BN, _IN_C)
    f2 = feats.reshape(_BN, _IN_C)
    r = lambda w: w.reshape(1, _OUT_C)
    y_pre, stats = _stage_b1(x2, fc1_w, r(fc1_b), r(ln1_g), r(ln1_b),
                             fc2_w, r(fc2_b), r(ln2_g), r(ln2_b),
                             r(dw_w), r(dw_b), pw_w, r(pw_b))
    out2 = _stage_b2(y_pre, stats, r(bn_g), r(bn_b), f2, proj_w, r(proj_b))
    return out2.reshape(_B, _N, _OUT_C)
